# baseline (speedup 1.0000x reference)
import functools

import jax
import jax.numpy as jnp
from jax import lax
from jax.experimental import pallas as pl
from jax.experimental.pallas import tpu as pltpu
from jax.experimental.pallas import tpu_sc as plsc

_LANES = 16
_NC, _NS = 2, 16
_NW = _NC * _NS


def _relayout(word_table):
    V, E = word_table.shape
    TC = V // 128
    TAIL = V - TC * 128
    base_per_w = TC // _NW
    rem = TC - base_per_w * _NW
    per_w = base_per_w + (1 if rem else 0)

    wtT = word_table.T
    tail64 = word_table[TC * 128:]

    @functools.partial(
        pl.kernel,
        mesh=plsc.VectorSubcoreMesh(core_axis_name="c", subcore_axis_name="s"),
        out_type=jax.ShapeDtypeStruct((V, 128), jnp.float32),
        compiler_params=pltpu.CompilerParams(
            use_tc_tiling_on_sc=True, needs_layout_passes=False),
        scratch_types=[
            pltpu.VMEM((2, E // 8, 8, 128), jnp.float32),
            pltpu.VMEM((128, 128), jnp.float32),
            pltpu.VMEM((128, 128), jnp.float32),
            pltpu.VMEM((64, E), jnp.float32),
            pltpu.SemaphoreType.DMA,
            pltpu.SemaphoreType.DMA,
            pltpu.SemaphoreType.DMA,
            pltpu.SemaphoreType.DMA,
        ],
    )
    def relayout_kernel(wtT_hbm, tail_hbm, out_hbm, tiles, rows0, rows1, tv,
                        sr0, sr1, sw0, sw1):
        sem_r = [sr0, sr1]
        sem_w = [sw0, sw1]
        rows = [rows0, rows1]
        wid = lax.axis_index("s") * _NC + lax.axis_index("c")
        start = wid * base_per_w + jnp.minimum(wid, rem)
        n_slots = per_w

        def slot(i):
            return jnp.minimum(start + i, TC - 1)

        def read(i, b):
            for c8 in range(E // 8):
                pltpu.async_copy(
                    wtT_hbm.at[pl.ds(8 * c8, 8), pl.ds(slot(i) * 128, 128)],
                    tiles.at[b, c8], sem_r[b])

        def wait_read(i, b):
            for c8 in range(E // 8):
                pltpu.make_async_copy(
                    wtT_hbm.at[pl.ds(8 * c8, 8), pl.ds(slot(i) * 128, 128)],
                    tiles.at[b, c8], sem_r[b]).wait()

        def write(i, b):
            pltpu.async_copy(rows[b], out_hbm.at[pl.ds(slot(i) * 128, 128)],
                             sem_w[b])

        def wait_write(i, b):
            pltpu.make_async_copy(rows[b],
                                  out_hbm.at[pl.ds(slot(i) * 128, 128)],
                                  sem_w[b]).wait()

        lanes = lax.iota(jnp.int32, _LANES)

        def transpose(b):
            def body(k, carry):
                row_idx = lanes + k * _LANES
                for c in range(E):
                    src = tiles[b, c // 8, c % 8, pl.ds(k * _LANES, _LANES)]
                    plsc.store_scatter(rows[b], [row_idx, lanes * 0 + c], src)
                return carry
            lax.fori_loop(0, 128 // _LANES, body, 0)

        read(0, 0)

        def step(s, carry):
            for b in range(2):
                i = s * 2 + b
                pl.when(i + 1 < n_slots)(lambda: read(i + 1, 1 - b))
                wait_read(i, b)
                pl.when(i >= 2)(lambda: wait_write(i - 2, b))
                transpose(b)
                write(i, b)
            return carry

        lax.fori_loop(0, n_slots // 2, step, 0)
        if n_slots % 2:
            i = n_slots - 1
            b = i % 2
            wait_read(i, b)
            pl.when(i >= 2)(lambda: wait_write(i - 2, b))
            transpose(b)
            write(i, b)
        for b in range(2):
            i = n_slots - 2 + b
            wait_write(i, i % 2)

        if TAIL:
            def tail():
                pltpu.async_copy(tail_hbm, tv, sem_r[0])
                pltpu.make_async_copy(tail_hbm, tv, sem_r[0]).wait()

                def body(r, carry):
                    for j in range(E // _LANES):
                        sl = pl.ds(j * _LANES, _LANES)
                        rows0[r, sl] = tv[r, sl]
                    return carry
                lax.fori_loop(0, TAIL, body, 0)
                pltpu.async_copy(rows0.at[pl.ds(0, TAIL)],
                                 out_hbm.at[pl.ds(TC * 128, TAIL)], sem_w[0])
                pltpu.make_async_copy(rows0.at[pl.ds(0, TAIL)],
                                      out_hbm.at[pl.ds(TC * 128, TAIL)],
                                      sem_w[0]).wait()
            pl.when(wid == 0)(tail)

    return relayout_kernel(wtT, tail64)


def _gather_add(ids_flat, wt_pad, pos_flat, N, E):
    CH = 200
    per_w = N // _NW
    n_chunks = per_w // CH

    wt128 = wt_pad

    @functools.partial(
        pl.kernel,
        mesh=plsc.VectorSubcoreMesh(core_axis_name="c", subcore_axis_name="s"),
        out_type=jax.ShapeDtypeStruct((N, E), jnp.float32),
        compiler_params=pltpu.CompilerParams(use_tc_tiling_on_sc=True),
        scratch_types=[
            pltpu.VMEM((2 * CH,), jnp.int32),
            pltpu.VMEM((2, CH, 128), jnp.float32),
            pltpu.VMEM((2, CH, E), jnp.float32),
            pltpu.VMEM((CH * E,), jnp.float32),
            pltpu.SemaphoreType.DMA,
            pltpu.SemaphoreType.DMA,
            pltpu.SemaphoreType.DMA,
            pltpu.SemaphoreType.DMA,
            pltpu.SemaphoreType.DMA,
            pltpu.SemaphoreType.DMA,
            pltpu.SemaphoreType.DMA,
        ],
    )
    def gather_kernel(ids_hbm, wt_hbm, pos_hbm, out_hbm,
                      idxr, gbuf, sbuf, pos_v,
                      si0, si1, sg0, sg1, sw0, sw1, sp):
        sem_i = [si0, si1]
        sem_g = [sg0, sg1]
        sem_w = [sw0, sw1]
        wid = lax.axis_index("s") * _NC + lax.axis_index("c")
        base = wid * per_w

        pltpu.async_copy(pos_hbm, pos_v, sp).wait()

        def load_idx(c, b):
            pltpu.async_copy(ids_hbm.at[pl.ds(base + c * CH, CH)],
                             idxr.at[pl.ds(b * CH, CH)], sem_i[b])

        def wait_idx(c, b):
            pltpu.make_async_copy(ids_hbm.at[pl.ds(base + c * CH, CH)],
                                  idxr.at[pl.ds(b * CH, CH)], sem_i[b]).wait()

        def gather(c, b):
            pltpu.async_copy(wt_hbm.at[idxr.at[pl.ds(b * CH, CH)]],
                             gbuf.at[b], sem_g[b])

        def wait_gather(c, b):
            pltpu.make_async_copy(wt_hbm.at[idxr.at[pl.ds(b * CH, CH)]],
                                  gbuf.at[b], sem_g[b]).wait()

        def write(c, b):
            pltpu.async_copy(sbuf.at[b],
                             out_hbm.at[pl.ds(base + c * CH, CH)], sem_w[b])

        def wait_write(c, b):
            pltpu.make_async_copy(sbuf.at[b],
                                  out_hbm.at[pl.ds(base + c * CH, CH)],
                                  sem_w[b]).wait()

        load_idx(0, 0)
        load_idx(1, 1)
        wait_idx(0, 0)
        gather(0, 0)

        def step(s, carry):
            for b in range(2):
                c = s * 2 + b
                def next_gather():
                    wait_idx(c + 1, 1 - b)
                    gather(c + 1, 1 - b)
                pl.when(c + 1 < n_chunks)(next_gather)
                wait_gather(c, b)
                pl.when(c >= 2)(lambda: wait_write(c - 2, b))

                def add_body(i, carry2):
                    for j in range(E // _LANES):
                        sl = pl.ds(j * _LANES, _LANES)
                        sbuf[b, i, sl] = (gbuf[b, i, sl]
                                          + pos_v[pl.ds(i * E + j * _LANES,
                                                        _LANES)])
                    return carry2
                lax.fori_loop(0, CH, add_body, 0)

                pl.when(c + 2 < n_chunks)(lambda: load_idx(c + 2, b))
                write(c, b)
            return carry

        lax.fori_loop(0, n_chunks // 2, step, 0)
        for b in range(2):
            wait_write(n_chunks - 2 + b, b)

    return gather_kernel(ids_flat, wt128, pos_flat)


def kernel(input_ids, word_table, pos_table):
    B, S = input_ids.shape
    V, E = word_table.shape
    N = B * S

    wt_pad = _relayout(word_table)
    ids_flat = input_ids.reshape(N).astype(jnp.int32)
    pos_flat = pos_table.reshape(S * E)
    out = _gather_add(ids_flat, wt_pad, pos_flat, N, E)
    return out.reshape(B, S, E)


# speedup vs baseline: 1.2475x; 1.2475x over previous
import functools

import jax
import jax.numpy as jnp
from jax import lax
from jax.experimental import pallas as pl
from jax.experimental.pallas import tpu as pltpu
from jax.experimental.pallas import tpu_sc as plsc

_LANES = 16
_NC, _NS = 2, 16
_NW = _NC * _NS


def _relayout(word_table):
    V, E = word_table.shape
    TC = V // 128
    TAIL = V - TC * 128
    base_per_w = TC // _NW
    rem = TC - base_per_w * _NW
    per_w = base_per_w + (1 if rem else 0)

    wtT = word_table.T
    tail64 = word_table[TC * 128:]

    @functools.partial(
        pl.kernel,
        mesh=plsc.VectorSubcoreMesh(core_axis_name="c", subcore_axis_name="s"),
        out_type=jax.ShapeDtypeStruct((V, 128), jnp.float32),
        compiler_params=pltpu.CompilerParams(
            use_tc_tiling_on_sc=True, needs_layout_passes=False),
        scratch_types=[
            pltpu.VMEM((E // 8, 8, 128), jnp.float32),
            pltpu.VMEM((E // 8, 8, 128), jnp.float32),
            pltpu.VMEM((128, 128), jnp.float32),
            pltpu.VMEM((128, 128), jnp.float32),
            pltpu.VMEM((64, E), jnp.float32),
            pltpu.SemaphoreType.DMA,
            pltpu.SemaphoreType.DMA,
            pltpu.SemaphoreType.DMA,
            pltpu.SemaphoreType.DMA,
        ],
    )
    def relayout_kernel(wtT_hbm, tail_hbm, out_hbm, tiles0, tiles1,
                        rows0, rows1, tv, sr0, sr1, sw0, sw1):
        sem_r = [sr0, sr1]
        sem_w = [sw0, sw1]
        tiles = [tiles0, tiles1]
        rows = [rows0, rows1]
        wid = lax.axis_index("s") * _NC + lax.axis_index("c")
        start = wid * base_per_w + jnp.minimum(wid, rem)
        n_slots = per_w

        def slot(i):
            return jnp.minimum(start + i, TC - 1)

        def read(i, b):
            for c8 in range(E // 8):
                pltpu.async_copy(
                    wtT_hbm.at[pl.ds(8 * c8, 8), pl.ds(slot(i) * 128, 128)],
                    tiles[b].at[c8], sem_r[b])

        def wait_read(i, b):
            for c8 in range(E // 8):
                pltpu.make_async_copy(
                    wtT_hbm.at[pl.ds(8 * c8, 8), pl.ds(slot(i) * 128, 128)],
                    tiles[b].at[c8], sem_r[b]).wait()

        def write(i, b):
            pltpu.async_copy(rows[b], out_hbm.at[pl.ds(slot(i) * 128, 128)],
                             sem_w[b])

        def wait_write(i, b):
            pltpu.make_async_copy(rows[b],
                                  out_hbm.at[pl.ds(slot(i) * 128, 128)],
                                  sem_w[b]).wait()

        lanes = lax.iota(jnp.int32, _LANES)
        rot = [(lanes + k) % _LANES for k in range(_LANES)]

        def transpose(b):
            def body(i, carry):
                row_idx = lanes + i * _LANES
                for c0 in range(0, E, _LANES):
                    for k in range(_LANES):
                        cvec = rot[k] + c0
                        val = plsc.load_gather(
                            tiles[b],
                            [lax.shift_right_logical(cvec, 2 + 1),
                             cvec & 7, row_idx])
                        plsc.store_scatter(rows[b], [row_idx, cvec], val)
                return carry
            lax.fori_loop(0, 128 // _LANES, body, 0)

        read(0, 0)

        def step(s, carry):
            for b in range(2):
                i = s * 2 + b
                pl.when(i + 1 < n_slots)(lambda: read(i + 1, 1 - b))
                wait_read(i, b)
                pl.when(i >= 2)(lambda: wait_write(i - 2, b))
                transpose(b)
                write(i, b)
            return carry

        lax.fori_loop(0, n_slots // 2, step, 0)
        if n_slots % 2:
            i = n_slots - 1
            b = i % 2
            wait_read(i, b)
            pl.when(i >= 2)(lambda: wait_write(i - 2, b))
            transpose(b)
            write(i, b)
        for b in range(2):
            i = n_slots - 2 + b
            wait_write(i, i % 2)

        if TAIL:
            def tail():
                pltpu.async_copy(tail_hbm, tv, sem_r[0])
                pltpu.make_async_copy(tail_hbm, tv, sem_r[0]).wait()

                def body(r, carry):
                    for j in range(E // _LANES):
                        sl = pl.ds(j * _LANES, _LANES)
                        rows0[r, sl] = tv[r, sl]
                    return carry
                lax.fori_loop(0, TAIL, body, 0)
                pltpu.async_copy(rows0.at[pl.ds(0, TAIL)],
                                 out_hbm.at[pl.ds(TC * 128, TAIL)], sem_w[0])
                pltpu.make_async_copy(rows0.at[pl.ds(0, TAIL)],
                                      out_hbm.at[pl.ds(TC * 128, TAIL)],
                                      sem_w[0]).wait()
            pl.when(wid == 0)(tail)

    return relayout_kernel(wtT, tail64)


def _gather_add(ids_flat, wt_pad, pos_flat, N, E):
    CH = 200
    per_w = N // _NW
    n_chunks = per_w // CH

    wt128 = wt_pad

    @functools.partial(
        pl.kernel,
        mesh=plsc.VectorSubcoreMesh(core_axis_name="c", subcore_axis_name="s"),
        out_type=jax.ShapeDtypeStruct((N, E), jnp.float32),
        compiler_params=pltpu.CompilerParams(use_tc_tiling_on_sc=True),
        scratch_types=[
            pltpu.VMEM((2 * CH,), jnp.int32),
            pltpu.VMEM((2, CH, 128), jnp.float32),
            pltpu.VMEM((2, CH, E), jnp.float32),
            pltpu.VMEM((CH * E,), jnp.float32),
            pltpu.SemaphoreType.DMA,
            pltpu.SemaphoreType.DMA,
            pltpu.SemaphoreType.DMA,
            pltpu.SemaphoreType.DMA,
            pltpu.SemaphoreType.DMA,
            pltpu.SemaphoreType.DMA,
            pltpu.SemaphoreType.DMA,
        ],
    )
    def gather_kernel(ids_hbm, wt_hbm, pos_hbm, out_hbm,
                      idxr, gbuf, sbuf, pos_v,
                      si0, si1, sg0, sg1, sw0, sw1, sp):
        sem_i = [si0, si1]
        sem_g = [sg0, sg1]
        sem_w = [sw0, sw1]
        wid = lax.axis_index("s") * _NC + lax.axis_index("c")
        base = wid * per_w

        pltpu.async_copy(pos_hbm, pos_v, sp).wait()

        def load_idx(c, b):
            pltpu.async_copy(ids_hbm.at[pl.ds(base + c * CH, CH)],
                             idxr.at[pl.ds(b * CH, CH)], sem_i[b])

        def wait_idx(c, b):
            pltpu.make_async_copy(ids_hbm.at[pl.ds(base + c * CH, CH)],
                                  idxr.at[pl.ds(b * CH, CH)], sem_i[b]).wait()

        def gather(c, b):
            pltpu.async_copy(wt_hbm.at[idxr.at[pl.ds(b * CH, CH)]],
                             gbuf.at[b], sem_g[b])

        def wait_gather(c, b):
            pltpu.make_async_copy(wt_hbm.at[idxr.at[pl.ds(b * CH, CH)]],
                                  gbuf.at[b], sem_g[b]).wait()

        def write(c, b):
            pltpu.async_copy(sbuf.at[b],
                             out_hbm.at[pl.ds(base + c * CH, CH)], sem_w[b])

        def wait_write(c, b):
            pltpu.make_async_copy(sbuf.at[b],
                                  out_hbm.at[pl.ds(base + c * CH, CH)],
                                  sem_w[b]).wait()

        load_idx(0, 0)
        load_idx(1, 1)
        wait_idx(0, 0)
        gather(0, 0)

        def step(s, carry):
            for b in range(2):
                c = s * 2 + b
                def next_gather():
                    wait_idx(c + 1, 1 - b)
                    gather(c + 1, 1 - b)
                pl.when(c + 1 < n_chunks)(next_gather)
                wait_gather(c, b)
                pl.when(c >= 2)(lambda: wait_write(c - 2, b))

                def add_body(i, carry2):
                    for dr in range(4):
                        r = i * 4 + dr
                        for j in range(E // _LANES):
                            sl = pl.ds(j * _LANES, _LANES)
                            sbuf[b, r, sl] = (gbuf[b, r, sl]
                                              + pos_v[pl.ds(r * E
                                                            + j * _LANES,
                                                            _LANES)])
                    return carry2
                lax.fori_loop(0, CH // 4, add_body, 0)

                pl.when(c + 2 < n_chunks)(lambda: load_idx(c + 2, b))
                write(c, b)
            return carry

        lax.fori_loop(0, n_chunks // 2, step, 0)
        for b in range(2):
            wait_write(n_chunks - 2 + b, b)

    return gather_kernel(ids_flat, wt128, pos_flat)


def kernel(input_ids, word_table, pos_table):
    B, S = input_ids.shape
    V, E = word_table.shape
    N = B * S

    wt_pad = _relayout(word_table)
    ids_flat = input_ids.reshape(N).astype(jnp.int32)
    pos_flat = pos_table.reshape(S * E)
    out = _gather_add(ids_flat, wt_pad, pos_flat, N, E)
    return out.reshape(B, S, E)
